# baseline (speedup 1.0000x reference)
import functools

import jax
import jax.numpy as jnp
from jax import lax
from jax.experimental import pallas as pl
from jax.experimental.pallas import tpu as pltpu
from jax.experimental.pallas import tpu_sc as plsc

N = 10000
E = 320000
NP = 10240
NC = 2
NS = 16
NW = NC * NS
EPW = E // NW
CH = 80
NCHUNK = EPW // CH
ROWS_PER_TILE = NP // NS

_mesh = plsc.VectorSubcoreMesh(core_axis_name="c", subcore_axis_name="s")


STRIPE = NP // NS
DCHUNKS = 2 * NCHUNK


@functools.partial(
    pl.kernel,
    mesh=_mesh,
    out_type=jax.ShapeDtypeStruct((NC, NP, 128), jnp.float32),
    scratch_types=[
        pltpu.VMEM((2, CH), jnp.int32),
        pltpu.VMEM((128, 128), jnp.float32),
        pltpu.VMEM((128, 128), jnp.float32),
        pltpu.VMEM_SHARED((NP, 128), jnp.float32),
        pltpu.SemaphoreType.DMA,
    ],
)
def _deg_kernel(e4d_hbm, ones_hbm, zeros_hbm, out_hbm, idxb, ob, zb, acc, isem):
    c = lax.axis_index("c")
    s = lax.axis_index("s")

    def echunk(k):
        return e4d_hbm.at[2 * s + k // NCHUNK, k % NCHUNK, c]

    pltpu.sync_copy(ones_hbm, ob)
    pltpu.sync_copy(zeros_hbm, zb)
    for t in range(STRIPE // 128):
        pltpu.sync_copy(zb, acc.at[pl.ds(s * STRIPE + t * 128, 128)])
    plsc.subcore_barrier()

    pltpu.async_copy(echunk(0), idxb.at[0], isem)

    def body(j, carry):
        b = lax.bitwise_and(j, 1)
        pltpu.make_async_copy(echunk(j), idxb.at[b], isem).wait()

        @pl.when(j + 1 < DCHUNKS)
        def _():
            pltpu.async_copy(echunk(j + 1), idxb.at[1 - b], isem)

        pltpu.sync_copy(ob.at[pl.ds(0, CH)], acc.at[idxb.at[b]], add=True)
        return carry
    lax.fori_loop(0, DCHUNKS, body, 0)

    plsc.subcore_barrier()
    pltpu.sync_copy(acc.at[pl.ds(s * STRIPE, STRIPE)],
                    out_hbm.at[c, pl.ds(s * STRIPE, STRIPE)])


@functools.partial(
    pl.kernel,
    mesh=_mesh,
    out_type=jax.ShapeDtypeStruct((NC, NP, 128), jnp.float32),
    scratch_types=[
        pltpu.VMEM((2, 2, CH), jnp.int32),
        pltpu.VMEM((2, 128, 128), jnp.float32),
        pltpu.VMEM_SHARED((NP, 128), jnp.float32),
        pltpu.SemaphoreType.DMA,
        pltpu.SemaphoreType.DMA,
    ],
)
def _agg_kernel(h_hbm, e_hbm, zeros_hbm, out_hbm, idxb, rows, acc, isem, gsem):
    c = lax.axis_index("c")
    s = lax.axis_index("s")
    w = c * NS + s

    pltpu.sync_copy(zeros_hbm, rows.at[0])
    for t in range(STRIPE // 128):
        pltpu.sync_copy(rows.at[0], acc.at[pl.ds(s * STRIPE + t * 128, 128)])

    plsc.subcore_barrier()

    pltpu.async_copy(e_hbm.at[w, 0], idxb.at[0], isem)
    pltpu.make_async_copy(e_hbm.at[w, 0], idxb.at[0], isem).wait()
    pltpu.async_copy(e_hbm.at[w, 1], idxb.at[1], isem)
    pltpu.async_copy(h_hbm.at[idxb.at[0, 0]], rows.at[0, pl.ds(0, CH)], gsem)

    def body(j, carry):
        b = lax.bitwise_and(j, 1)
        pltpu.make_async_copy(h_hbm.at[idxb.at[b, 0]],
                              rows.at[b, pl.ds(0, CH)], gsem).wait()

        @pl.when(j + 1 < NCHUNK)
        def _():
            pltpu.make_async_copy(e_hbm.at[w, j + 1], idxb.at[1 - b], isem).wait()
            pltpu.async_copy(h_hbm.at[idxb.at[1 - b, 0]],
                             rows.at[1 - b, pl.ds(0, CH)], gsem)

        pltpu.sync_copy(rows.at[b, pl.ds(0, CH)], acc.at[idxb.at[b, 1]], add=True)

        @pl.when(j + 2 < NCHUNK)
        def _():
            pltpu.async_copy(e_hbm.at[w, j + 2], idxb.at[b], isem)
        return carry
    lax.fori_loop(0, NCHUNK, body, 0)

    plsc.subcore_barrier()
    pltpu.sync_copy(acc.at[pl.ds(s * STRIPE, STRIPE)],
                    out_hbm.at[c, pl.ds(s * STRIPE, STRIPE)])


_R = 2000
_G = N // _R


def _norm(d):
    return lax.rsqrt(jnp.maximum(d, 1.0))


def _mm1_body(f_ref, w_ref, d_ref, o_ref):
    o_ref[...] = jnp.dot(f_ref[...], w_ref[...],
                         preferred_element_type=jnp.float32) * _norm(d_ref[...])


def _mm1(feats, W1, dsrc):
    return pl.pallas_call(
        _mm1_body,
        grid=(_G,),
        in_specs=[
            pl.BlockSpec((_R, 128), lambda i: (i, 0)),
            pl.BlockSpec((128, 128), lambda i: (0, 0)),
            pl.BlockSpec((_R, 1), lambda i: (i, 0)),
        ],
        out_specs=pl.BlockSpec((_R, 128), lambda i: (i, 0)),
        out_shape=jax.ShapeDtypeStruct((N, 128), jnp.float32),
    )(feats, W1, dsrc)


def _mm2_body(p_ref, dd_ref, b_ref, w_ref, ds_ref, o_ref):
    x = (p_ref[0] + p_ref[1]) * _norm(dd_ref[...]) + b_ref[...]
    x = jnp.maximum(x, 0.0)
    o_ref[...] = jnp.dot(x, w_ref[...],
                         preferred_element_type=jnp.float32) * _norm(ds_ref[...])


def _mm2(parts, ddst, b1r, W2, dsrc):
    return pl.pallas_call(
        _mm2_body,
        grid=(_G,),
        in_specs=[
            pl.BlockSpec((NC, _R, 128), lambda i: (0, i, 0)),
            pl.BlockSpec((_R, 1), lambda i: (i, 0)),
            pl.BlockSpec((1, 128), lambda i: (0, 0)),
            pl.BlockSpec((128, 128), lambda i: (0, 0)),
            pl.BlockSpec((_R, 1), lambda i: (i, 0)),
        ],
        out_specs=pl.BlockSpec((_R, 128), lambda i: (i, 0)),
        out_shape=jax.ShapeDtypeStruct((N, 128), jnp.float32),
    )(parts, ddst, b1r, W2, dsrc)


def _final_body(p_ref, dd_ref, b_ref, w_ref, bp_ref, o_ref):
    x = (p_ref[0] + p_ref[1]) * _norm(dd_ref[...]) + b_ref[...]
    x = jnp.maximum(x, 0.0)
    lg = jnp.dot(x, w_ref[...], preferred_element_type=jnp.float32) + bp_ref[...]
    m = jnp.max(lg, axis=1, keepdims=True)
    e = jnp.exp(lg - m)
    o_ref[...] = e / jnp.sum(e, axis=1, keepdims=True)


def _final(parts, ddst, b2r, Wpp, bpp):
    return pl.pallas_call(
        _final_body,
        grid=(_G,),
        in_specs=[
            pl.BlockSpec((NC, _R, 128), lambda i: (0, i, 0)),
            pl.BlockSpec((_R, 1), lambda i: (i, 0)),
            pl.BlockSpec((1, 128), lambda i: (0, 0)),
            pl.BlockSpec((128, 128), lambda i: (0, 0)),
            pl.BlockSpec((1, 128), lambda i: (0, 0)),
        ],
        out_specs=pl.BlockSpec((_R, 128), lambda i: (i, 0)),
        out_shape=jax.ShapeDtypeStruct((N, 128), jnp.float32),
    )(parts, ddst, b2r, Wpp, bpp)


def kernel(features, edge_index, edge_types, W1, b1, W2, b2, Wp, bp):
    L = Wp.shape[1]
    e4d = jnp.stack(
        [edge_index[0].astype(jnp.int32).reshape(NW, NCHUNK, CH),
         edge_index[1].astype(jnp.int32).reshape(NW, NCHUNK, CH)],
        axis=2)

    ones_r = jnp.ones((128, 128), jnp.float32)
    zrows = jnp.zeros((128, 128), jnp.float32)
    deg = _deg_kernel(e4d, ones_r, zrows)
    dsrc = deg[0, :, 0].reshape(NP, 1)
    ddst = deg[1, :, 0].reshape(NP, 1)

    b1r = b1.reshape(1, 128)
    b2r = b2.reshape(1, 128)
    Wpp = jnp.pad(Wp, ((0, 0), (0, 128 - L)))
    bpp = jnp.pad(bp, (0, 128 - L), constant_values=-1e30).reshape(1, 128)

    h1 = _mm1(features, W1, dsrc)
    p1 = _agg_kernel(h1, e4d, zrows)
    h2 = _mm2(p1, ddst, b1r, W2, dsrc)
    p2 = _agg_kernel(h2, e4d, zrows)
    out = _final(p2, ddst, b2r, Wpp, bpp)
    return out[:, :L]


# speedup vs baseline: 1.0490x; 1.0490x over previous
import functools

import jax
import jax.numpy as jnp
from jax import lax
from jax.experimental import pallas as pl
from jax.experimental.pallas import tpu as pltpu
from jax.experimental.pallas import tpu_sc as plsc

N = 10000
E = 320000
NP = 10240
NC = 2
NS = 16
NW = NC * NS
EPW = E // NW
CH = 80
NCHUNK = EPW // CH
STRIPE = NP // NS
DCHUNKS = E // CH // NS

_mesh = plsc.VectorSubcoreMesh(core_axis_name="c", subcore_axis_name="s")


@functools.partial(
    pl.kernel,
    mesh=_mesh,
    out_type=jax.ShapeDtypeStruct((NC, NP, 128), jnp.float32),
    scratch_types=[
        pltpu.VMEM((2, CH), jnp.int32),
        pltpu.VMEM((128, 128), jnp.float32),
        pltpu.VMEM_SHARED((NP, 128), jnp.float32),
        pltpu.SemaphoreType.DMA,
    ],
)
def _deg_kernel(ei2_hbm, out_hbm, idxb, ob, acc, isem):
    c = lax.axis_index("c")
    s = lax.axis_index("s")

    def zrow(r, carry):
        for k in range(8):
            ob[r, pl.ds(k * 16, 16)] = jnp.zeros((16,), jnp.float32)
        return carry
    lax.fori_loop(0, 128, zrow, 0)
    for t in range(STRIPE // 128):
        pltpu.sync_copy(ob, acc.at[pl.ds(s * STRIPE + t * 128, 128)])

    def orow(r, carry):
        for k in range(8):
            ob[r, pl.ds(k * 16, 16)] = jnp.ones((16,), jnp.float32)
        return carry
    lax.fori_loop(0, 128, orow, 0)
    plsc.subcore_barrier()

    base = s * DCHUNKS
    pltpu.async_copy(ei2_hbm.at[c, base], idxb.at[0], isem)

    def body(j, carry):
        b = lax.bitwise_and(j, 1)
        pltpu.make_async_copy(ei2_hbm.at[c, base + j], idxb.at[b], isem).wait()

        @pl.when(j + 1 < DCHUNKS)
        def _():
            pltpu.async_copy(ei2_hbm.at[c, base + j + 1], idxb.at[1 - b], isem)

        pltpu.sync_copy(ob.at[pl.ds(0, CH)], acc.at[idxb.at[b]], add=True)
        return carry
    lax.fori_loop(0, DCHUNKS, body, 0)

    plsc.subcore_barrier()
    pltpu.sync_copy(acc.at[pl.ds(s * STRIPE, STRIPE)],
                    out_hbm.at[c, pl.ds(s * STRIPE, STRIPE)])


@functools.partial(
    pl.kernel,
    mesh=_mesh,
    out_type=jax.ShapeDtypeStruct((NC, NP, 128), jnp.float32),
    scratch_types=[
        pltpu.VMEM((2, 2, CH), jnp.int32),
        pltpu.VMEM((2, CH, 128), jnp.float32),
        pltpu.VMEM_SHARED((NP, 128), jnp.float32),
        pltpu.SemaphoreType.DMA,
        pltpu.SemaphoreType.DMA,
    ],
)
def _agg_kernel(h_hbm, e_hbm, out_hbm, idxb, rows, acc, isem, gsem):
    c = lax.axis_index("c")
    s = lax.axis_index("s")
    w = c * NS + s

    def zrow(r, carry):
        for k in range(8):
            rows[0, r, pl.ds(k * 16, 16)] = jnp.zeros((16,), jnp.float32)
        return carry
    lax.fori_loop(0, CH, zrow, 0)
    for t in range(STRIPE // CH):
        pltpu.sync_copy(rows.at[0], acc.at[pl.ds(s * STRIPE + t * CH, CH)])

    plsc.subcore_barrier()

    pltpu.async_copy(e_hbm.at[w, 0], idxb.at[0], isem)
    pltpu.make_async_copy(e_hbm.at[w, 0], idxb.at[0], isem).wait()
    pltpu.async_copy(e_hbm.at[w, 1], idxb.at[1], isem)
    pltpu.async_copy(h_hbm.at[idxb.at[0, 0]], rows.at[0], gsem)

    def body(j, carry):
        b = lax.bitwise_and(j, 1)
        pltpu.make_async_copy(h_hbm.at[idxb.at[b, 0]], rows.at[b], gsem).wait()

        @pl.when(j + 1 < NCHUNK)
        def _():
            pltpu.make_async_copy(e_hbm.at[w, j + 1], idxb.at[1 - b], isem).wait()
            pltpu.async_copy(h_hbm.at[idxb.at[1 - b, 0]], rows.at[1 - b], gsem)

        pltpu.sync_copy(rows.at[b], acc.at[idxb.at[b, 1]], add=True)

        @pl.when(j + 2 < NCHUNK)
        def _():
            pltpu.async_copy(e_hbm.at[w, j + 2], idxb.at[b], isem)
        return carry
    lax.fori_loop(0, NCHUNK, body, 0)

    plsc.subcore_barrier()
    pltpu.sync_copy(acc.at[pl.ds(s * STRIPE, STRIPE)],
                    out_hbm.at[c, pl.ds(s * STRIPE, STRIPE)])


_R = 2560
_G = NP // _R


def _mm1_body(f_ref, w_ref, n_ref, o_ref):
    o_ref[...] = jnp.dot(f_ref[...], w_ref[...],
                         preferred_element_type=jnp.float32) * n_ref[...]


def _mm1(fpad, W1, nsrc):
    return pl.pallas_call(
        _mm1_body,
        grid=(_G,),
        in_specs=[
            pl.BlockSpec((_R, 128), lambda i: (i, 0)),
            pl.BlockSpec((128, 128), lambda i: (0, 0)),
            pl.BlockSpec((_R, 1), lambda i: (i, 0)),
        ],
        out_specs=pl.BlockSpec((_R, 128), lambda i: (i, 0)),
        out_shape=jax.ShapeDtypeStruct((NP, 128), jnp.float32),
    )(fpad, W1, nsrc)


def _mm2_body(p_ref, nd_ref, b_ref, w_ref, ns_ref, o_ref):
    x = (p_ref[0] + p_ref[1]) * nd_ref[...] + b_ref[...]
    x = jnp.maximum(x, 0.0)
    o_ref[...] = jnp.dot(x, w_ref[...],
                         preferred_element_type=jnp.float32) * ns_ref[...]


def _mm2(parts, ndst, b1r, W2, nsrc):
    return pl.pallas_call(
        _mm2_body,
        grid=(_G,),
        in_specs=[
            pl.BlockSpec((NC, _R, 128), lambda i: (0, i, 0)),
            pl.BlockSpec((_R, 1), lambda i: (i, 0)),
            pl.BlockSpec((1, 128), lambda i: (0, 0)),
            pl.BlockSpec((128, 128), lambda i: (0, 0)),
            pl.BlockSpec((_R, 1), lambda i: (i, 0)),
        ],
        out_specs=pl.BlockSpec((_R, 128), lambda i: (i, 0)),
        out_shape=jax.ShapeDtypeStruct((NP, 128), jnp.float32),
    )(parts, ndst, b1r, W2, nsrc)


def _final_body(p_ref, nd_ref, b_ref, w_ref, bp_ref, o_ref):
    x = (p_ref[0] + p_ref[1]) * nd_ref[...] + b_ref[...]
    x = jnp.maximum(x, 0.0)
    lg = jnp.dot(x, w_ref[...], preferred_element_type=jnp.float32) + bp_ref[...]
    m = jnp.max(lg, axis=1, keepdims=True)
    e = jnp.exp(lg - m)
    o_ref[...] = e / jnp.sum(e, axis=1, keepdims=True)


def _final(parts, ndst, b2r, Wpp, bpp):
    return pl.pallas_call(
        _final_body,
        grid=(_G,),
        in_specs=[
            pl.BlockSpec((NC, _R, 128), lambda i: (0, i, 0)),
            pl.BlockSpec((_R, 1), lambda i: (i, 0)),
            pl.BlockSpec((1, 128), lambda i: (0, 0)),
            pl.BlockSpec((128, 128), lambda i: (0, 0)),
            pl.BlockSpec((1, 128), lambda i: (0, 0)),
        ],
        out_specs=pl.BlockSpec((_R, 128), lambda i: (i, 0)),
        out_shape=jax.ShapeDtypeStruct((NP, 128), jnp.float32),
    )(parts, ndst, b2r, Wpp, bpp)


def kernel(features, edge_index, edge_types, W1, b1, W2, b2, Wp, bp):
    L = Wp.shape[1]
    e4d = jnp.stack(
        [edge_index[0].astype(jnp.int32).reshape(NW, NCHUNK, CH),
         edge_index[1].astype(jnp.int32).reshape(NW, NCHUNK, CH)],
        axis=2)

    ei2 = jnp.stack([edge_index[0].astype(jnp.int32).reshape(E // CH, CH),
                     edge_index[1].astype(jnp.int32).reshape(E // CH, CH)])
    deg = _deg_kernel(ei2)[:, :, 0]
    nsrc = lax.rsqrt(jnp.maximum(deg[0], 1.0)).reshape(NP, 1)
    ndst = lax.rsqrt(jnp.maximum(deg[1], 1.0)).reshape(NP, 1)

    fpad = jnp.pad(features, ((0, NP - N), (0, 0)))
    b1r = b1.reshape(1, 128)
    b2r = b2.reshape(1, 128)
    Wpp = jnp.pad(Wp, ((0, 0), (0, 128 - L)))
    bpp = jnp.pad(bp, (0, 128 - L), constant_values=-1e30).reshape(1, 128)

    h1 = _mm1(fpad, W1, nsrc)
    p1 = _agg_kernel(h1, e4d)
    h2 = _mm2(p1, ndst, b1r, W2, nsrc)
    p2 = _agg_kernel(h2, e4d)
    out = _final(p2, ndst, b2r, Wpp, bpp)
    return out[:N, :L]
